# baseline (speedup 1.0000x reference)
import functools

import jax
import jax.numpy as jnp
from jax.experimental import pallas as pl
from jax.experimental.pallas import tpu as pltpu

_N_MAX_PIXELS = 20000
_THRESHOLD = 0.5


def _body(x_ref, out_ref, m_ref, carry_ref, *, block_pixels):
    i = pl.program_id(1)

    @pl.when(i == 0)
    def _():
        carry_ref[0] = 0

    x = x_ref[0]
    f = (x[:, 0:1] > _THRESHOLD).astype(jnp.int32)
    s = jnp.sum(f)
    carry = carry_ref[0]

    def fast(f, carry):
        keep_all = carry + s <= _N_MAX_PIXELS
        return jnp.where(keep_all, f, 0).astype(x.dtype)

    def slow(f, carry):
        cum = f
        d = 1
        while d < block_pixels:
            shifted = jnp.concatenate(
                [jnp.zeros((d, 1), jnp.int32), cum[: block_pixels - d]], axis=0
            )
            cum = cum + shifted
            d *= 2
        keep = jnp.logical_and(f == 1, carry + cum <= _N_MAX_PIXELS)
        return keep.astype(x.dtype)

    pred = jnp.logical_or(carry + s <= _N_MAX_PIXELS, carry >= _N_MAX_PIXELS)
    m = jax.lax.cond(pred, fast, slow, f, carry)

    m_ref[0] = m
    out_ref[0] = x * m
    carry_ref[0] = carry + s


def _pick_block(hw):
    for p in (2048, 1024, 512, 256, 128, 64, 32, 16, 8):
        if hw % p == 0:
            return p
    return hw


def kernel(inputs):
    b, h, w, c = inputs.shape
    hw = h * w
    p = _pick_block(hw)
    x = inputs.reshape(b, hw, c)
    grid = (b, hw // p)

    out, mask = pl.pallas_call(
        functools.partial(_body, block_pixels=p),
        grid=grid,
        in_specs=[pl.BlockSpec((1, p, c), lambda bi, i: (bi, i, 0))],
        out_specs=[
            pl.BlockSpec((1, p, c), lambda bi, i: (bi, i, 0)),
            pl.BlockSpec((1, p, 1), lambda bi, i: (bi, i, 0)),
        ],
        out_shape=[
            jax.ShapeDtypeStruct((b, hw, c), inputs.dtype),
            jax.ShapeDtypeStruct((b, hw, 1), inputs.dtype),
        ],
        scratch_shapes=[pltpu.SMEM((1,), jnp.int32)],
        compiler_params=pltpu.CompilerParams(
            dimension_semantics=("arbitrary", "arbitrary")
        ),
    )(x)

    return out.reshape(b, h, w, c), mask.reshape(b, h, w, 1)


# speedup vs baseline: 1.0646x; 1.0646x over previous
import functools

import jax
import jax.numpy as jnp
from jax.experimental import pallas as pl
from jax.experimental.pallas import tpu as pltpu

_N_MAX_PIXELS = 20000
_THRESHOLD = 0.5


def _body(x_ref, out_ref, m_ref, carry_ref, *, block_pixels):
    i = pl.program_id(1)

    @pl.when(i == 0)
    def _():
        carry_ref[0] = 0

    x = x_ref[0]
    f = (x[:, 0:1] > _THRESHOLD).astype(jnp.float32)
    s = jnp.sum(f).astype(jnp.int32)
    carry = carry_ref[0]

    keep_all = (carry + s <= _N_MAX_PIXELS).astype(jnp.float32)
    m = f * keep_all
    m_ref[0] = m
    out_ref[0] = x * m

    @pl.when(jnp.logical_and(carry + s > _N_MAX_PIXELS, carry < _N_MAX_PIXELS))
    def _():
        cum = f
        d = 1
        while d < block_pixels:
            shifted = jnp.concatenate(
                [jnp.zeros((d, 1), jnp.float32), cum[: block_pixels - d]], axis=0
            )
            cum = cum + shifted
            d *= 2
        limit = (_N_MAX_PIXELS - carry) + 0.5
        mb = f * (cum < limit).astype(jnp.float32)
        m_ref[0] = mb
        out_ref[0] = x * mb

    carry_ref[0] = carry + s


def _pick_block(hw):
    for p in (4096, 2048, 1024, 512, 256, 128, 64, 32, 16, 8):
        if hw % p == 0:
            return p
    return hw


def kernel(inputs):
    b, h, w, c = inputs.shape
    hw = h * w
    p = _pick_block(hw)
    x = inputs.reshape(b, hw, c)
    grid = (b, hw // p)

    out, mask = pl.pallas_call(
        functools.partial(_body, block_pixels=p),
        grid=grid,
        in_specs=[pl.BlockSpec((1, p, c), lambda bi, i: (bi, i, 0))],
        out_specs=[
            pl.BlockSpec((1, p, c), lambda bi, i: (bi, i, 0)),
            pl.BlockSpec((1, p, 1), lambda bi, i: (bi, i, 0)),
        ],
        out_shape=[
            jax.ShapeDtypeStruct((b, hw, c), inputs.dtype),
            jax.ShapeDtypeStruct((b, hw, 1), inputs.dtype),
        ],
        scratch_shapes=[pltpu.SMEM((1,), jnp.int32)],
        compiler_params=pltpu.CompilerParams(
            dimension_semantics=("arbitrary", "arbitrary")
        ),
    )(x)

    return out.reshape(b, h, w, c), mask.reshape(b, h, w, 1)
